# baseline (speedup 1.0000x reference)
import functools

import jax
import jax.numpy as jnp
from jax import lax
from jax.experimental import pallas as pl
from jax.experimental.pallas import tpu as pltpu
from jax.experimental.pallas import tpu_sc as plsc

B, C, E, P = 32, 16, 128, 512
NUM_PERM = 1000

NC, NS, L = 2, 16, 16
NW = NC * NS
PAIRS_PER_W = (B * C) // NW
CE = 32
NCH = E // CE
NU = PAIRS_PER_W * NCH


def _shuffle_gather(x3, order):
    mesh = plsc.VectorSubcoreMesh(core_axis_name="c", subcore_axis_name="s")

    @functools.partial(
        pl.kernel,
        out_type=jax.ShapeDtypeStruct((B * C, E, P), jnp.float32),
        mesh=mesh,
        compiler_params=pltpu.CompilerParams(needs_layout_passes=False),
        scratch_types=[
            pltpu.VMEM((P,), jnp.int32),
            pltpu.VMEM((CE, P), jnp.float32),
            pltpu.VMEM((CE, P), jnp.float32),
            pltpu.VMEM((CE, P), jnp.float32),
            pltpu.VMEM((CE, P), jnp.float32),
            pltpu.SemaphoreType.DMA,
            pltpu.SemaphoreType.DMA,
            pltpu.SemaphoreType.DMA,
            pltpu.SemaphoreType.DMA,
        ],
    )
    def k(x_hbm, idx_hbm, out_hbm, idxbuf, in0, in1, out0, out1,
          isem0, isem1, osem0, osem1):
        cid = lax.axis_index("c")
        sid = lax.axis_index("s")
        w = sid * NC + cid
        c = w // 2
        bh = w % 2
        inb = (in0, in1)
        outb = (out0, out1)
        isem = (isem0, isem1)
        osem = (osem0, osem1)

        pltpu.sync_copy(idx_hbm.at[c], idxbuf)

        def unit_slot(u):
            kk = u // NCH
            ch = u % NCH
            b = bh * (B // 2) + kk
            return b * C + c, ch * CE

        def in_copy(u, par):
            pair, e0 = unit_slot(u)
            return pltpu.make_async_copy(
                x_hbm.at[pair, pl.ds(e0, CE), :], inb[par], isem[par])

        def out_copy(u, par):
            pair, e0 = unit_slot(u)
            return pltpu.make_async_copy(
                outb[par], out_hbm.at[pair, pl.ds(e0, CE), :], osem[par])

        def compute(inbuf, outbuf):
            @plsc.parallel_loop(0, P // L, unroll=8)
            def j_body(j):
                iv = idxbuf[pl.ds(j * L, L)]

                @plsc.parallel_loop(0, CE, unroll=16)
                def e_body(e):
                    ev = jnp.full((L,), e, dtype=jnp.int32)
                    outbuf[e, pl.ds(j * L, L)] = plsc.load_gather(
                        inbuf, [ev, iv])

        in_copy(0, 0).start()

        def outer(i, _):
            for par in range(2):
                u = i * 2 + par

                @pl.when(u + 1 < NU)
                def _start_next():
                    in_copy(u + 1, 1 - par).start()

                in_copy(u, par).wait()

                @pl.when(u >= 2)
                def _drain_prev():
                    out_copy(u - 2, par).wait()

                compute(inb[par], outb[par])
                out_copy(u, par).start()
            return 0

        lax.fori_loop(0, NU // 2, outer, 0)
        out_copy(NU - 2, 0).wait()
        out_copy(NU - 1, 1).wait()

    return k(x3, order)


def kernel(X, shuffled_idx):
    rand_idx = jax.random.randint(jax.random.key(1), (1,), 0, NUM_PERM - 1)[0]
    order = lax.dynamic_index_in_dim(
        shuffled_idx, rand_idx, axis=0, keepdims=False
    ).astype(jnp.int32)
    x3 = X.reshape(B * C, E, P)
    out = _shuffle_gather(x3, order)
    return out.reshape(B, C, E, P)


# speedup vs baseline: 1.2746x; 1.2746x over previous
import functools

import jax
import jax.numpy as jnp
from jax import lax
from jax.experimental import pallas as pl
from jax.experimental.pallas import tpu as pltpu
from jax.experimental.pallas import tpu_sc as plsc

B, C, E, P = 32, 16, 128, 512
NUM_PERM = 1000

NC, NS, L = 2, 16, 16
NW = NC * NS
PAIRS_PER_W = (B * C) // NW
CE = 32
NCH = E // CE
NU = PAIRS_PER_W * NCH


def _shuffle_gather(x3, order):
    mesh = plsc.VectorSubcoreMesh(core_axis_name="c", subcore_axis_name="s")

    @functools.partial(
        pl.kernel,
        out_type=jax.ShapeDtypeStruct((B * C, E, P), jnp.float32),
        mesh=mesh,
        compiler_params=pltpu.CompilerParams(needs_layout_passes=False),
        scratch_types=[
            pltpu.VMEM((P,), jnp.int32),
            pltpu.VMEM((CE, P), jnp.float32),
            pltpu.VMEM((CE, P), jnp.float32),
            pltpu.VMEM((CE, P), jnp.float32),
            pltpu.VMEM((CE, P), jnp.float32),
            pltpu.SemaphoreType.DMA,
            pltpu.SemaphoreType.DMA,
            pltpu.SemaphoreType.DMA,
            pltpu.SemaphoreType.DMA,
        ],
    )
    def k(x_hbm, idx_hbm, out_hbm, idxbuf, in0, in1, out0, out1,
          isem0, isem1, osem0, osem1):
        cid = lax.axis_index("c")
        sid = lax.axis_index("s")
        w = sid * NC + cid
        c = w // 2
        bh = w % 2
        inb = (in0, in1)
        outb = (out0, out1)
        isem = (isem0, isem1)
        osem = (osem0, osem1)

        pltpu.sync_copy(idx_hbm.at[c], idxbuf)

        def unit_slot(u):
            kk = u // NCH
            ch = u % NCH
            b = bh * (B // 2) + kk
            return b * C + c, ch * CE

        def in_copy(u, par):
            pair, e0 = unit_slot(u)
            return pltpu.make_async_copy(
                x_hbm.at[pair, pl.ds(e0, CE), :], inb[par], isem[par])

        def out_copy(u, par):
            pair, e0 = unit_slot(u)
            return pltpu.make_async_copy(
                outb[par], out_hbm.at[pair, pl.ds(e0, CE), :], osem[par])

        def compute(inbuf, outbuf):
            @plsc.parallel_loop(0, P // L, unroll=8)
            def j_body(j):
                iv = idxbuf[pl.ds(j * L, L)]

                @plsc.parallel_loop(0, CE, unroll=8)
                def e_body(e):
                    ev = jnp.full((L,), e, dtype=jnp.int32)
                    outbuf[e, pl.ds(j * L, L)] = plsc.load_gather(
                        inbuf, [ev, iv])

        in_copy(0, 0).start()

        def outer(i, _):
            for par in range(2):
                u = i * 2 + par

                @pl.when(u + 1 < NU)
                def _start_next():
                    in_copy(u + 1, 1 - par).start()

                in_copy(u, par).wait()

                @pl.when(u >= 2)
                def _drain_prev():
                    out_copy(u - 2, par).wait()

                compute(inb[par], outb[par])
                out_copy(u, par).start()
            return 0

        lax.fori_loop(0, NU // 2, outer, 0)
        out_copy(NU - 2, 0).wait()
        out_copy(NU - 1, 1).wait()

    return k(x3, order)


def kernel(X, shuffled_idx):
    rand_idx = jax.random.randint(jax.random.key(1), (1,), 0, NUM_PERM - 1)[0]
    order = lax.dynamic_index_in_dim(
        shuffled_idx, rand_idx, axis=0, keepdims=False
    ).astype(jnp.int32)
    x3 = X.reshape(B * C, E, P)
    out = _shuffle_gather(x3, order)
    return out.reshape(B, C, E, P)
